# Initial kernel scaffold; baseline (speedup 1.0000x reference)
import jax
import jax.numpy as jnp
from jax.experimental import pallas as pl


def kernel(x, edge_index, edge_attr, batch, atom_table, bond_table, W_init, b_init, W_upd, b_upd, W1, b1, W2, b2):
    raise NotImplementedError("write your pallas kernel here")


# speedup vs baseline: 5.3814x; 5.3814x over previous
import functools

import jax
import jax.numpy as jnp
from jax import lax
from jax.experimental import pallas as pl
from jax.experimental.pallas import tpu as pltpu
from jax.experimental.pallas import tpu_sc as plsc

N = 50000
E = 800000
NGRAPH = 1024

NC = 2
NS = 16
L = 16

K = 256
G = 128
KG = K // G
CHUNKS = 196
EPT = K * CHUNKS
E_PAD = EPT * NS
NPT = 3200
N_PAD = NPT * NS
G_PAD = NGRAPH + 8
T_PAD = 480

_mesh = plsc.VectorSubcoreMesh(
    core_axis_name="c", subcore_axis_name="s", num_cores=NC, num_subcores=NS)
_sc_params = pltpu.CompilerParams(use_tc_tiling_on_sc=False)


def _zero_rows(rows_ref, nrows):
    zero = jnp.zeros((L,), jnp.float32)

    def body(i, _):
        for u in range(8):
            r = i * 8 + u
            rows_ref[r, pl.ds(0, L)] = zero
            rows_ref[r, pl.ds(L, L)] = zero
        return 0

    lax.fori_loop(0, nrows // 8, body, 0)


def _sc_pass0(x_hbm, src_hbm, dst_hbm, ea_hbm, tbl_hbm,
              c_hbm, agg_hbm,
              acc, xs, tbl, sv, dv, cv, eav, xg, rows, sem, sem2):
    cid = lax.axis_index("c")
    sid = lax.axis_index("s")

    _zero_rows(rows, G)

    def zbody(j, _):
        pltpu.sync_copy(rows.at[pl.ds(0, G)], acc.at[pl.ds(sid * NPT + j * G, G)])
        return 0

    lax.fori_loop(0, NPT // G, zbody, 0)
    pltpu.sync_copy(x_hbm.at[pl.ds(sid * NPT, NPT)], xs.at[pl.ds(sid * NPT, NPT)])

    @pl.when(sid == 0)
    def _():
        pltpu.sync_copy(tbl_hbm.at[cid], tbl)

    plsc.subcore_barrier()

    def chunk(ich, _):
        row0 = sid * (CHUNKS * KG) + ich * KG
        pltpu.sync_copy(src_hbm.at[pl.ds(row0, KG)], sv)
        pltpu.sync_copy(ea_hbm.at[pl.ds(row0, KG)], eav)
        descs = [pltpu.async_copy(xs.at[sv.at[j]], xg.at[j], sem)
                 for j in range(KG)]
        for d in descs:
            d.wait()
        for j in range(KG):
            for k in range(G // L):
                s = pl.ds(k * L, L)
                cv[j, s] = xg[j, s] * 4 + eav[j, s]
        pltpu.sync_copy(cv, c_hbm.at[pl.ds(row0, KG)])
        descs = [pltpu.async_copy(tbl.at[cv.at[j]],
                                  rows.at[pl.ds(j * G, G)], sem2)
                 for j in range(KG)]
        for d in descs:
            d.wait()
        pltpu.sync_copy(dst_hbm.at[pl.ds(row0, KG)], dv)
        for j in range(KG):
            pltpu.sync_copy(rows.at[pl.ds(j * G, G)], acc.at[dv.at[j]], add=True)
        return 0

    lax.fori_loop(0, CHUNKS, chunk, 0)
    plsc.subcore_barrier()
    pltpu.sync_copy(acc.at[pl.ds(sid * NPT, NPT)], agg_hbm.at[cid, sid])


def _sc_pass_body(src_hbm, dst_hbm, c_hbm, aggw0_hbm, aggw1_hbm, tbl_hbm,
                  acc, tbl, sv, dv, cv, rows, brows, sem, sem2,
                  cid, sid):
    _zero_rows(rows, G)

    def zbody(j, _):
        pltpu.sync_copy(rows.at[pl.ds(0, G)], acc.at[pl.ds(sid * NPT + j * G, G)])
        return 0

    lax.fori_loop(0, NPT // G, zbody, 0)

    @pl.when(sid == 0)
    def _():
        pltpu.sync_copy(tbl_hbm.at[cid], tbl)

    plsc.subcore_barrier()

    def chunk(ich, _):
        row0 = sid * (CHUNKS * KG) + ich * KG
        pltpu.sync_copy(src_hbm.at[pl.ds(row0, KG)], sv)
        pltpu.sync_copy(c_hbm.at[pl.ds(row0, KG)], cv)

        @pl.when(cid == 0)
        def _():
            descs = [pltpu.async_copy(aggw0_hbm.at[sv.at[j]],
                                      rows.at[pl.ds(j * G, G)], sem)
                     for j in range(KG)]
            for d in descs:
                d.wait()

        @pl.when(cid == 1)
        def _():
            descs = [pltpu.async_copy(aggw1_hbm.at[sv.at[j]],
                                      rows.at[pl.ds(j * G, G)], sem)
                     for j in range(KG)]
            for d in descs:
                d.wait()

        descs = [pltpu.async_copy(tbl.at[cv.at[j]],
                                  brows.at[pl.ds(j * G, G)], sem2)
                 for j in range(KG)]
        for d in descs:
            d.wait()

        def cbody(i, _):
            for u in range(8):
                r = i * 8 + u
                for h in (0, L):
                    s = pl.ds(h, L)
                    rows[r, s] = jnp.maximum(rows[r, s] + brows[r, s], 0.0)
            return 0

        lax.fori_loop(0, K // 8, cbody, 0)

        pltpu.sync_copy(dst_hbm.at[pl.ds(row0, KG)], dv)
        for j in range(KG):
            pltpu.sync_copy(rows.at[pl.ds(j * G, G)], acc.at[dv.at[j]], add=True)
        return 0

    lax.fori_loop(0, CHUNKS, chunk, 0)
    plsc.subcore_barrier()


def _sc_pass(src_hbm, dst_hbm, c_hbm, aggw0_hbm, aggw1_hbm, tbl_hbm,
             agg_hbm,
             acc, tbl, sv, dv, cv, rows, brows, sem, sem2):
    cid = lax.axis_index("c")
    sid = lax.axis_index("s")
    _sc_pass_body(src_hbm, dst_hbm, c_hbm, aggw0_hbm, aggw1_hbm, tbl_hbm,
                  acc, tbl, sv, dv, cv, rows, brows, sem, sem2, cid, sid)
    pltpu.sync_copy(acc.at[pl.ds(sid * NPT, NPT)], agg_hbm.at[cid, sid])


def _sc_pass3(src_hbm, dst_hbm, c_hbm, aggw0_hbm, aggw1_hbm, tbl_hbm, batch_hbm,
              mol_hbm,
              acc, tbl, mol, sv, dv, cv, rows, brows, sem, sem2):
    cid = lax.axis_index("c")
    sid = lax.axis_index("s")
    _zero_rows(rows, G)

    @pl.when(sid < 8)
    def _():
        pltpu.sync_copy(rows.at[pl.ds(0, G)], mol.at[pl.ds(sid * G, G)])

    @pl.when(sid == 0)
    def _():
        pltpu.sync_copy(rows.at[pl.ds(0, 8)], mol.at[pl.ds(NGRAPH, 8)])

    _sc_pass_body(src_hbm, dst_hbm, c_hbm, aggw0_hbm, aggw1_hbm, tbl_hbm,
                  acc, tbl, sv, dv, cv, rows, brows, sem, sem2, cid, sid)

    def rbody(g, _):
        node0 = sid * NPT + g * G
        brow = sid * (NPT // G) + g
        pltpu.sync_copy(batch_hbm.at[brow], dv.at[0])
        pltpu.sync_copy(acc.at[pl.ds(node0, G)], rows.at[pl.ds(0, G)])
        pltpu.sync_copy(rows.at[pl.ds(0, G)], mol.at[dv.at[0]], add=True)
        return 0

    lax.fori_loop(0, NPT // G, rbody, 0)
    plsc.subcore_barrier()

    @pl.when(sid == 0)
    def _():
        pltpu.sync_copy(mol, mol_hbm.at[cid])


def _tc_prep(at_ref, bt_ref, wi_ref, wu_ref, bi_ref, bu_ref,
             aw_ref, bw_ref, au_ref, bu_out_ref):
    at = at_ref[...]
    bt = bt_ref[...]
    aw_ref[...] = jnp.dot(at, wi_ref[0:64, :], preferred_element_type=jnp.float32)
    bw_ref[...] = jnp.dot(bt, wi_ref[64:80, :], preferred_element_type=jnp.float32) + bi_ref[...]
    au_ref[...] = jnp.dot(at, wu_ref[0:64, :], preferred_element_type=jnp.float32)
    bu_out_ref[...] = jnp.dot(bt, wu_ref[64:80, :], preferred_element_type=jnp.float32) + bu_ref[...]


def _tc_matmul(agg_ref, wm_ref, o0_ref, o1_ref):
    a0 = agg_ref[0]
    a1 = agg_ref[1]
    o0_ref[...] = (jnp.dot(a0, wm_ref[0:32, 0:32], preferred_element_type=jnp.float32)
                   + jnp.dot(a1, wm_ref[32:64, 0:32], preferred_element_type=jnp.float32))
    o1_ref[...] = (jnp.dot(a0, wm_ref[0:32, 32:64], preferred_element_type=jnp.float32)
                   + jnp.dot(a1, wm_ref[32:64, 32:64], preferred_element_type=jnp.float32))


def _tc_readout(mol_ref, w1_ref, b1_ref, w2_ref, b2_ref, out_ref):
    m = jnp.concatenate([mol_ref[0], mol_ref[1]], axis=1)
    h = jnp.maximum(jnp.dot(m, w1_ref[...], preferred_element_type=jnp.float32)
                    + b1_ref[...], 0.0)
    o = jnp.sum(h * w2_ref[...], axis=1, keepdims=True) + b2_ref[...]
    out_ref[...] = o[:NGRAPH]


def kernel(x, edge_index, edge_attr, batch, atom_table, bond_table,
           W_init, b_init, W_upd, b_upd, W1, b1, W2, b2):
    f32 = jnp.float32
    src = edge_index[0]
    dst = edge_index[1]

    pe = E_PAD - E
    src_p = jnp.concatenate([src, jnp.arange(pe, dtype=jnp.int32) % N])
    dst_p = jnp.concatenate([dst, N + jnp.arange(pe, dtype=jnp.int32) % (N_PAD - N)])
    ea_p = jnp.concatenate([edge_attr, jnp.zeros((pe,), jnp.int32)])
    src2 = src_p.reshape(E_PAD // G, G)
    dst2 = dst_p.reshape(E_PAD // G, G)
    ea2 = ea_p.reshape(E_PAD // G, G)
    x_p = jnp.concatenate([x, jnp.zeros((N_PAD - N,), jnp.int32)])
    batch_p = jnp.concatenate(
        [batch, NGRAPH + jnp.arange(N_PAD - N, dtype=jnp.int32) % (G_PAD - NGRAPH)])
    batch2 = batch_p.reshape(N_PAD // G, G)

    aw, bw, au, bu = pl.pallas_call(
        _tc_prep,
        out_shape=[jax.ShapeDtypeStruct((119, 64), f32),
                   jax.ShapeDtypeStruct((4, 64), f32),
                   jax.ShapeDtypeStruct((119, 64), f32),
                   jax.ShapeDtypeStruct((4, 64), f32)],
    )(atom_table, bond_table, W_init, W_upd, b_init.reshape(1, 64), b_upd.reshape(1, 64))

    def halves(t):
        t = jnp.concatenate([t, jnp.zeros((T_PAD - 476, 64), f32)])
        return jnp.stack([t[:, :32], t[:, 32:]])

    initT = halves(jax.nn.relu(aw[:, None, :] + bw[None, :, :]).reshape(476, 64))
    baseT = halves((au[:, None, :] + bu[None, :, :]).reshape(476, 64))
    Wm = W_upd[80:]

    pass0 = pl.kernel(
        _sc_pass0,
        out_type=[jax.ShapeDtypeStruct((E_PAD // G, G), jnp.int32),
                  jax.ShapeDtypeStruct((NC, NS, NPT, 32), f32)],
        mesh=_mesh,
        compiler_params=_sc_params,
        scratch_types=[
            pltpu.VMEM_SHARED((N_PAD, 32), f32),
            pltpu.VMEM_SHARED((N_PAD,), jnp.int32),
            pltpu.VMEM_SHARED((T_PAD, 32), f32),
            pltpu.VMEM((KG, G), jnp.int32),
            pltpu.VMEM((KG, G), jnp.int32),
            pltpu.VMEM((KG, G), jnp.int32),
            pltpu.VMEM((KG, G), jnp.int32),
            pltpu.VMEM((KG, G), jnp.int32),
            pltpu.VMEM((K, 32), f32),
            pltpu.SemaphoreType.DMA,
            pltpu.SemaphoreType.DMA,
        ],
    )
    c2, agg = pass0(x_p, src2, dst2, ea2, initT)

    sc_pass = pl.kernel(
        _sc_pass,
        out_type=jax.ShapeDtypeStruct((NC, NS, NPT, 32), f32),
        mesh=_mesh,
        compiler_params=_sc_params,
        scratch_types=[
            pltpu.VMEM_SHARED((N_PAD, 32), f32),
            pltpu.VMEM_SHARED((T_PAD, 32), f32),
            pltpu.VMEM((KG, G), jnp.int32),
            pltpu.VMEM((KG, G), jnp.int32),
            pltpu.VMEM((KG, G), jnp.int32),
            pltpu.VMEM((K, 32), f32),
            pltpu.VMEM((K, 32), f32),
            pltpu.SemaphoreType.DMA,
            pltpu.SemaphoreType.DMA,
        ],
    )
    sc_pass3 = pl.kernel(
        _sc_pass3,
        out_type=jax.ShapeDtypeStruct((NC, G_PAD, 32), f32),
        mesh=_mesh,
        compiler_params=_sc_params,
        scratch_types=[
            pltpu.VMEM_SHARED((N_PAD, 32), f32),
            pltpu.VMEM_SHARED((T_PAD, 32), f32),
            pltpu.VMEM_SHARED((G_PAD, 32), f32),
            pltpu.VMEM((KG, G), jnp.int32),
            pltpu.VMEM((KG, G), jnp.int32),
            pltpu.VMEM((KG, G), jnp.int32),
            pltpu.VMEM((K, 32), f32),
            pltpu.VMEM((K, 32), f32),
            pltpu.SemaphoreType.DMA,
            pltpu.SemaphoreType.DMA,
        ],
    )

    tc_matmul = pl.pallas_call(
        _tc_matmul,
        grid=(8,),
        in_specs=[
            pl.BlockSpec((NC, N_PAD // 8, 32), lambda i: (0, i, 0)),
            pl.BlockSpec((64, 64), lambda i: (0, 0)),
        ],
        out_specs=[
            pl.BlockSpec((N_PAD // 8, 32), lambda i: (i, 0)),
            pl.BlockSpec((N_PAD // 8, 32), lambda i: (i, 0)),
        ],
        out_shape=[jax.ShapeDtypeStruct((N_PAD, 32), f32),
                   jax.ShapeDtypeStruct((N_PAD, 32), f32)],
    )

    for p in range(3):
        aggw0, aggw1 = tc_matmul(agg.reshape(NC, N_PAD, 32), Wm)
        if p < 2:
            agg = sc_pass(src2, dst2, c2, aggw0, aggw1, baseT)
        else:
            mol = sc_pass3(src2, dst2, c2, aggw0, aggw1, baseT, batch2)

    out = pl.pallas_call(
        _tc_readout,
        out_shape=jax.ShapeDtypeStruct((NGRAPH, 1), f32),
    )(mol, W1, b1.reshape(1, 64), W2.reshape(1, 64), b2.reshape(1, 1))
    return out.reshape(NGRAPH)


# speedup vs baseline: 10.7747x; 2.0022x over previous
import jax
import jax.numpy as jnp
from jax import lax
from jax.experimental import pallas as pl
from jax.experimental.pallas import tpu as pltpu
from jax.experimental.pallas import tpu_sc as plsc

N = 50000
E = 800000
NGRAPH = 1024

NC = 2
NS = 16
L = 16

G = 128
CHUNKS = 392
EPT = G * CHUNKS
E_PAD = EPT * NS
NPT = 3200
N_PAD = NPT * NS
G_PAD = NGRAPH + 8
T_PAD = 480

_mesh = plsc.VectorSubcoreMesh(
    core_axis_name="c", subcore_axis_name="s", num_cores=NC, num_subcores=NS)
_sc_params = pltpu.CompilerParams(use_tc_tiling_on_sc=False)


def _wait(src, dst, sem):
    pltpu.make_async_copy(src, dst, sem).wait()


def _zero_rows(rows_ref, nrows):
    zero = jnp.zeros((L,), jnp.float32)

    def body(i, _):
        for u in range(8):
            r = i * 8 + u
            rows_ref[r, pl.ds(0, L)] = zero
            rows_ref[r, pl.ds(L, L)] = zero
        return 0

    lax.fori_loop(0, nrows // 8, body, 0)


def _zero_acc(rows, acc, sid):
    _zero_rows(rows, G)

    def zbody(j, _):
        pltpu.sync_copy(rows.at[pl.ds(0, G)], acc.at[pl.ds(sid * NPT + j * G, G)])
        return 0

    lax.fori_loop(0, NPT // G, zbody, 0)


def _sc_pass0(x_hbm, src_hbm, dst_hbm, ea_hbm, tbl_hbm,
              c_hbm, agg_hbm,
              acc, xs, tbl,
              sv0, sv1, ea0, ea1, dv0, dv1, cv0, cv1, xg0, xg1,
              rows0, rows1,
              sem_i, sem_x, sem_d, sem_b, sem_s, sem_cw):
    cid = lax.axis_index("c")
    sid = lax.axis_index("s")
    svs, eas, dvs, cvs, xgs, rows = ((sv0, sv1), (ea0, ea1), (dv0, dv1),
                                     (cv0, cv1), (xg0, xg1), (rows0, rows1))

    _zero_acc(rows0, acc, sid)
    pltpu.sync_copy(x_hbm.at[pl.ds(sid * NPT, NPT)], xs.at[pl.ds(sid * NPT, NPT)])

    @pl.when(sid == 0)
    def _():
        pltpu.sync_copy(tbl_hbm.at[cid], tbl)

    plsc.subcore_barrier()

    base_row = sid * CHUNKS

    pltpu.async_copy(src_hbm.at[pl.ds(base_row, 1)], sv0, sem_i)
    pltpu.async_copy(ea_hbm.at[pl.ds(base_row, 1)], ea0, sem_i)
    pltpu.async_copy(src_hbm.at[pl.ds(base_row + 1, 1)], sv1, sem_i)
    pltpu.async_copy(ea_hbm.at[pl.ds(base_row + 1, 1)], ea1, sem_i)
    _wait(src_hbm.at[pl.ds(0, 1)], sv0, sem_i)
    _wait(ea_hbm.at[pl.ds(0, 1)], ea0, sem_i)
    pltpu.async_copy(xs.at[sv0.at[0]], xg0.at[0], sem_x)

    def sub(i, p):
        q = 1 - p
        svp, svq, eap, dvp, dvq = svs[p], svs[q], eas[p], dvs[p], dvs[q]
        cvp, xgp, xgq, rp, rq = cvs[p], xgs[p], xgs[q], rows[p], rows[q]

        @pl.when(i >= 2)
        def _():
            _wait(tbl_hbm.at[0, pl.ds(0, G)], acc.at[pl.ds(0, G)], sem_s)
            _wait(cvp, c_hbm.at[pl.ds(0, 1)], sem_cw)

        pltpu.async_copy(dst_hbm.at[pl.ds(base_row + i, 1)], dvp, sem_d)

        @pl.when(i + 1 < CHUNKS)
        def _():
            _wait(src_hbm.at[pl.ds(0, 1)], svq, sem_i)
            _wait(ea_hbm.at[pl.ds(0, 1)], eas[q], sem_i)
            pltpu.async_copy(xs.at[svq.at[0]], xgq.at[0], sem_x)

        _wait(x_hbm.at[pl.ds(0, G)], xgp.at[0], sem_x)
        for k in range(G // L):
            s = pl.ds(k * L, L)
            cvp[0, s] = xgp[0, s] * 4 + eap[0, s]
        pltpu.async_copy(cvp, c_hbm.at[pl.ds(base_row + i, 1)], sem_cw)
        pltpu.async_copy(tbl.at[cvp.at[0]], rp, sem_b)

        @pl.when(i >= 1)
        def _():
            _wait(tbl_hbm.at[0, pl.ds(0, G)], rq, sem_b)
            _wait(dst_hbm.at[pl.ds(0, 1)], dvq, sem_d)
            pltpu.async_copy(rq, acc.at[dvq.at[0]], sem_s, add=True)

        @pl.when(i + 2 < CHUNKS)
        def _():
            pltpu.async_copy(src_hbm.at[pl.ds(base_row + i + 2, 1)], svp, sem_i)
            pltpu.async_copy(ea_hbm.at[pl.ds(base_row + i + 2, 1)], eap, sem_i)

    def outer(t, _):
        sub(2 * t, 0)
        sub(2 * t + 1, 1)
        return 0

    lax.fori_loop(0, CHUNKS // 2, outer, 0)

    lastp = (CHUNKS - 1) % 2
    _wait(tbl_hbm.at[0, pl.ds(0, G)], rows[lastp], sem_b)
    _wait(dst_hbm.at[pl.ds(0, 1)], dvs[lastp], sem_d)
    pltpu.async_copy(rows[lastp], acc.at[dvs[lastp].at[0]], sem_s, add=True)
    for p in (0, 1):
        _wait(tbl_hbm.at[0, pl.ds(0, G)], acc.at[pl.ds(0, G)], sem_s)
        _wait(cvs[p], c_hbm.at[pl.ds(0, 1)], sem_cw)

    plsc.subcore_barrier()
    pltpu.sync_copy(acc.at[pl.ds(sid * NPT, NPT)], agg_hbm.at[cid, sid])


def _sc_pass_body(src_hbm, dst_hbm, c_hbm, aggw0_hbm, aggw1_hbm, tbl_hbm,
                  acc, tbl,
                  sv0, sv1, dv0, dv1, cv0, cv1,
                  rows0, rows1, brows0, brows1, sbuf0, sbuf1,
                  sem_i, sem_r, sem_b, sem_d, sem_s,
                  cid, sid):
    svs, dvs, cvs = (sv0, sv1), (dv0, dv1), (cv0, cv1)
    rows, brows, sbufs = (rows0, rows1), (brows0, brows1), (sbuf0, sbuf1)

    _zero_acc(rows0, acc, sid)

    @pl.when(sid == 0)
    def _():
        pltpu.sync_copy(tbl_hbm.at[cid], tbl)

    plsc.subcore_barrier()

    base_row = sid * CHUNKS

    def fire_gather(sv_ref, rows_ref):
        @pl.when(cid == 0)
        def _():
            pltpu.async_copy(aggw0_hbm.at[sv_ref.at[0]], rows_ref, sem_r)

        @pl.when(cid == 1)
        def _():
            pltpu.async_copy(aggw1_hbm.at[sv_ref.at[0]], rows_ref, sem_r)

    pltpu.async_copy(src_hbm.at[pl.ds(base_row, 1)], sv0, sem_i)
    pltpu.async_copy(c_hbm.at[pl.ds(base_row, 1)], cv0, sem_i)
    pltpu.async_copy(src_hbm.at[pl.ds(base_row + 1, 1)], sv1, sem_i)
    pltpu.async_copy(c_hbm.at[pl.ds(base_row + 1, 1)], cv1, sem_i)
    _wait(src_hbm.at[pl.ds(0, 1)], sv0, sem_i)
    _wait(c_hbm.at[pl.ds(0, 1)], cv0, sem_i)
    fire_gather(sv0, rows0)
    pltpu.async_copy(tbl.at[cv0.at[0]], brows0, sem_b)

    def sub(i, p):
        q = 1 - p
        svp, svq, dvp, cvq = svs[p], svs[q], dvs[p], cvs[q]
        rp, rq, bp, bq, sp = rows[p], rows[q], brows[p], brows[q], sbufs[p]

        @pl.when(i >= 2)
        def _():
            _wait(aggw0_hbm.at[pl.ds(0, G)], acc.at[pl.ds(0, G)], sem_s)

        pltpu.async_copy(dst_hbm.at[pl.ds(base_row + i, 1)], dvp, sem_d)

        @pl.when(i + 1 < CHUNKS)
        def _():
            _wait(src_hbm.at[pl.ds(0, 1)], svq, sem_i)
            _wait(c_hbm.at[pl.ds(0, 1)], cvq, sem_i)
            fire_gather(svq, rq)
            pltpu.async_copy(tbl.at[cvq.at[0]], bq, sem_b)

        _wait(aggw0_hbm.at[pl.ds(0, G)], rp, sem_r)
        _wait(aggw0_hbm.at[pl.ds(0, G)], bp, sem_b)

        def cbody(k, _):
            for u in range(8):
                r = k * 8 + u
                for h in (0, L):
                    s = pl.ds(h, L)
                    sp[r, s] = jnp.maximum(rp[r, s] + bp[r, s], 0.0)
            return 0

        lax.fori_loop(0, G // 8, cbody, 0)

        _wait(dst_hbm.at[pl.ds(0, 1)], dvp, sem_d)
        pltpu.async_copy(sp, acc.at[dvp.at[0]], sem_s, add=True)

        @pl.when(i + 2 < CHUNKS)
        def _():
            pltpu.async_copy(src_hbm.at[pl.ds(base_row + i + 2, 1)], svp, sem_i)
            pltpu.async_copy(c_hbm.at[pl.ds(base_row + i + 2, 1)], cvs[p], sem_i)

    def outer(t, _):
        sub(2 * t, 0)
        sub(2 * t + 1, 1)
        return 0

    lax.fori_loop(0, CHUNKS // 2, outer, 0)
    for p in (0, 1):
        _wait(aggw0_hbm.at[pl.ds(0, G)], acc.at[pl.ds(0, G)], sem_s)
    plsc.subcore_barrier()


def _sc_pass(src_hbm, dst_hbm, c_hbm, aggw0_hbm, aggw1_hbm, tbl_hbm,
             agg_hbm,
             acc, tbl, sv0, sv1, dv0, dv1, cv0, cv1,
             rows0, rows1, brows0, brows1, sbuf0, sbuf1,
             sem_i, sem_r, sem_b, sem_d, sem_s):
    cid = lax.axis_index("c")
    sid = lax.axis_index("s")
    _sc_pass_body(src_hbm, dst_hbm, c_hbm, aggw0_hbm, aggw1_hbm, tbl_hbm,
                  acc, tbl, sv0, sv1, dv0, dv1, cv0, cv1,
                  rows0, rows1, brows0, brows1, sbuf0, sbuf1,
                  sem_i, sem_r, sem_b, sem_d, sem_s, cid, sid)
    pltpu.sync_copy(acc.at[pl.ds(sid * NPT, NPT)], agg_hbm.at[cid, sid])


def _sc_pass3(src_hbm, dst_hbm, c_hbm, aggw0_hbm, aggw1_hbm, tbl_hbm, batch_hbm,
              mol_hbm,
              acc, tbl, mol, sv0, sv1, dv0, dv1, cv0, cv1,
              rows0, rows1, brows0, brows1, sbuf0, sbuf1,
              sem_i, sem_r, sem_b, sem_d, sem_s):
    cid = lax.axis_index("c")
    sid = lax.axis_index("s")
    _zero_rows(rows0, G)

    @pl.when(sid < 8)
    def _():
        pltpu.sync_copy(rows0.at[pl.ds(0, G)], mol.at[pl.ds(sid * G, G)])

    @pl.when(sid == 0)
    def _():
        pltpu.sync_copy(rows0.at[pl.ds(0, 8)], mol.at[pl.ds(NGRAPH, 8)])

    _sc_pass_body(src_hbm, dst_hbm, c_hbm, aggw0_hbm, aggw1_hbm, tbl_hbm,
                  acc, tbl, sv0, sv1, dv0, dv1, cv0, cv1,
                  rows0, rows1, brows0, brows1, sbuf0, sbuf1,
                  sem_i, sem_r, sem_b, sem_d, sem_s, cid, sid)

    def rbody(g, _):
        node0 = sid * NPT + g * G
        brow = sid * (NPT // G) + g
        pltpu.sync_copy(batch_hbm.at[brow], dv0.at[0])
        pltpu.sync_copy(acc.at[pl.ds(node0, G)], rows0)
        pltpu.sync_copy(rows0, mol.at[dv0.at[0]], add=True)
        return 0

    lax.fori_loop(0, NPT // G, rbody, 0)
    plsc.subcore_barrier()

    @pl.when(sid == 0)
    def _():
        pltpu.sync_copy(mol, mol_hbm.at[cid])


def _tc_prep(at_ref, bt_ref, wi_ref, wu_ref, bi_ref, bu_ref,
             aw_ref, bw_ref, au_ref, bu_out_ref):
    at = at_ref[...]
    bt = bt_ref[...]
    aw_ref[...] = jnp.dot(at, wi_ref[0:64, :], preferred_element_type=jnp.float32)
    bw_ref[...] = jnp.dot(bt, wi_ref[64:80, :], preferred_element_type=jnp.float32) + bi_ref[...]
    au_ref[...] = jnp.dot(at, wu_ref[0:64, :], preferred_element_type=jnp.float32)
    bu_out_ref[...] = jnp.dot(bt, wu_ref[64:80, :], preferred_element_type=jnp.float32) + bu_ref[...]


def _tc_matmul(agg_ref, wm_ref, o0_ref, o1_ref):
    a0 = agg_ref[0]
    a1 = agg_ref[1]
    o0_ref[...] = (jnp.dot(a0, wm_ref[0:32, 0:32], preferred_element_type=jnp.float32)
                   + jnp.dot(a1, wm_ref[32:64, 0:32], preferred_element_type=jnp.float32))
    o1_ref[...] = (jnp.dot(a0, wm_ref[0:32, 32:64], preferred_element_type=jnp.float32)
                   + jnp.dot(a1, wm_ref[32:64, 32:64], preferred_element_type=jnp.float32))


def _tc_readout(mol_ref, w1_ref, b1_ref, w2_ref, b2_ref, out_ref):
    m = jnp.concatenate([mol_ref[0], mol_ref[1]], axis=1)
    h = jnp.maximum(jnp.dot(m, w1_ref[...], preferred_element_type=jnp.float32)
                    + b1_ref[...], 0.0)
    o = jnp.sum(h * w2_ref[...], axis=1, keepdims=True) + b2_ref[...]
    out_ref[...] = o[:NGRAPH]


def kernel(x, edge_index, edge_attr, batch, atom_table, bond_table,
           W_init, b_init, W_upd, b_upd, W1, b1, W2, b2):
    f32 = jnp.float32
    src = edge_index[0]
    dst = edge_index[1]

    pe = E_PAD - E
    src_p = jnp.concatenate([src, jnp.arange(pe, dtype=jnp.int32) % N])
    dst_p = jnp.concatenate([dst, N + jnp.arange(pe, dtype=jnp.int32) % (N_PAD - N)])
    ea_p = jnp.concatenate([edge_attr, jnp.zeros((pe,), jnp.int32)])
    src2 = src_p.reshape(E_PAD // G, G)
    dst2 = dst_p.reshape(E_PAD // G, G)
    ea2 = ea_p.reshape(E_PAD // G, G)
    x_p = jnp.concatenate([x, jnp.zeros((N_PAD - N,), jnp.int32)])
    batch_p = jnp.concatenate(
        [batch, NGRAPH + jnp.arange(N_PAD - N, dtype=jnp.int32) % (G_PAD - NGRAPH)])
    batch2 = batch_p.reshape(N_PAD // G, G)

    aw, bw, au, bu = pl.pallas_call(
        _tc_prep,
        out_shape=[jax.ShapeDtypeStruct((119, 64), f32),
                   jax.ShapeDtypeStruct((4, 64), f32),
                   jax.ShapeDtypeStruct((119, 64), f32),
                   jax.ShapeDtypeStruct((4, 64), f32)],
    )(atom_table, bond_table, W_init, W_upd, b_init.reshape(1, 64), b_upd.reshape(1, 64))

    def halves(t):
        t = jnp.concatenate([t, jnp.zeros((T_PAD - 476, 64), f32)])
        return jnp.stack([t[:, :32], t[:, 32:]])

    initT = halves(jax.nn.relu(aw[:, None, :] + bw[None, :, :]).reshape(476, 64))
    baseT = halves((au[:, None, :] + bu[None, :, :]).reshape(476, 64))
    Wm = W_upd[80:]

    idx2 = lambda: pltpu.VMEM((1, G), jnp.int32)
    rbuf = lambda: pltpu.VMEM((G, 32), f32)

    pass0 = pl.kernel(
        _sc_pass0,
        out_type=[jax.ShapeDtypeStruct((E_PAD // G, G), jnp.int32),
                  jax.ShapeDtypeStruct((NC, NS, NPT, 32), f32)],
        mesh=_mesh,
        compiler_params=_sc_params,
        scratch_types=[
            pltpu.VMEM_SHARED((N_PAD, 32), f32),
            pltpu.VMEM_SHARED((N_PAD,), jnp.int32),
            pltpu.VMEM_SHARED((T_PAD, 32), f32),
            idx2(), idx2(), idx2(), idx2(), idx2(),
            idx2(), idx2(), idx2(), idx2(), idx2(),
            rbuf(), rbuf(),
            pltpu.SemaphoreType.DMA, pltpu.SemaphoreType.DMA,
            pltpu.SemaphoreType.DMA, pltpu.SemaphoreType.DMA,
            pltpu.SemaphoreType.DMA, pltpu.SemaphoreType.DMA,
        ],
    )
    c2, agg = pass0(x_p, src2, dst2, ea2, initT)

    pass_scratch = [
        pltpu.VMEM_SHARED((N_PAD, 32), f32),
        pltpu.VMEM_SHARED((T_PAD, 32), f32),
        idx2(), idx2(), idx2(), idx2(), idx2(), idx2(),
        rbuf(), rbuf(), rbuf(), rbuf(), rbuf(), rbuf(),
        pltpu.SemaphoreType.DMA, pltpu.SemaphoreType.DMA,
        pltpu.SemaphoreType.DMA, pltpu.SemaphoreType.DMA,
        pltpu.SemaphoreType.DMA,
    ]
    sc_pass = pl.kernel(
        _sc_pass,
        out_type=jax.ShapeDtypeStruct((NC, NS, NPT, 32), f32),
        mesh=_mesh,
        compiler_params=_sc_params,
        scratch_types=list(pass_scratch),
    )
    sc_pass3 = pl.kernel(
        _sc_pass3,
        out_type=jax.ShapeDtypeStruct((NC, G_PAD, 32), f32),
        mesh=_mesh,
        compiler_params=_sc_params,
        scratch_types=(pass_scratch[:1]
                       + [pltpu.VMEM_SHARED((T_PAD, 32), f32),
                          pltpu.VMEM_SHARED((G_PAD, 32), f32)]
                       + pass_scratch[2:]),
    )

    tc_matmul = pl.pallas_call(
        _tc_matmul,
        grid=(8,),
        in_specs=[
            pl.BlockSpec((NC, N_PAD // 8, 32), lambda i: (0, i, 0)),
            pl.BlockSpec((64, 64), lambda i: (0, 0)),
        ],
        out_specs=[
            pl.BlockSpec((N_PAD // 8, 32), lambda i: (i, 0)),
            pl.BlockSpec((N_PAD // 8, 32), lambda i: (i, 0)),
        ],
        out_shape=[jax.ShapeDtypeStruct((N_PAD, 32), f32),
                   jax.ShapeDtypeStruct((N_PAD, 32), f32)],
    )

    for p in range(3):
        aggw0, aggw1 = tc_matmul(agg.reshape(NC, N_PAD, 32), Wm)
        if p < 2:
            agg = sc_pass(src2, dst2, c2, aggw0, aggw1, baseT)
        else:
            mol = sc_pass3(src2, dst2, c2, aggw0, aggw1, baseT, batch2)

    out = pl.pallas_call(
        _tc_readout,
        out_shape=jax.ShapeDtypeStruct((NGRAPH, 1), f32),
    )(mol, W1, b1.reshape(1, 64), W2.reshape(1, 64), b2.reshape(1, 1))
    return out.reshape(NGRAPH)
